# baseline (speedup 1.0000x reference)
import functools

import jax
import jax.numpy as jnp
from jax import lax
from jax.experimental import pallas as pl
from jax.experimental.pallas import tpu as pltpu
from jax.experimental.pallas import tpu_sc as plsc

_L = 200
_B = 4096
_V = 1_000_000
_D = 64

_NC, _NS = 2, 16
_NW = _NC * _NS
_BPW = _B // _NW
_LANES = 16

_CHUNK = 320
_SHARD = 31264
_NCHUNKS = 98
_GROUPS = _CHUNK // _LANES


def _sc_mv_body(emb_hbm, w_hbm, s_hbm, w_v, buf0, buf1, shard_v,
                sem0, sem1):
    wid = lax.axis_index("s") * _NC + lax.axis_index("c")
    base_row = (_V // _NW) * wid // 16 * 16

    pltpu.sync_copy(w_hbm, w_v)

    def chunk_off(k):
        return jnp.where(k < _NCHUNKS - 1, k * _CHUNK, _SHARD - _CHUNK)

    pltpu.async_copy(emb_hbm.at[pl.ds(base_row * _D, _CHUNK * _D)], buf0, sem0)
    pltpu.async_copy(emb_hbm.at[pl.ds((base_row + _CHUNK) * _D, _CHUNK * _D)],
                     buf1, sem1)

    wq = [w_v[pl.ds(q * _LANES, _LANES)] for q in range(4)]
    lane = jnp.arange(16, dtype=jnp.int32)
    zero = jnp.zeros((_LANES,), jnp.float32)

    def compute(buf, off):
        def rbody(r, res):
            base = r * _D
            t = buf[pl.ds(base, _LANES)] * wq[0]
            for q in range(1, 4):
                t = t + buf[pl.ds(base + q * _LANES, _LANES)] * wq[q]
            for sh in (8, 4, 2, 1):
                perm = (lane + sh) % _LANES
                t = t + lax.gather(
                    t, perm[:, None],
                    lax.GatherDimensionNumbers(
                        offset_dims=(), collapsed_slice_dims=(0,),
                        start_index_map=(0,)),
                    slice_sizes=(1,),
                    mode=lax.GatherScatterMode.PROMISE_IN_BOUNDS)
            j = lax.rem(r, _LANES)
            res = jnp.where(lane == j, t, res)

            @pl.when(j == _LANES - 1)
            def _():
                shard_v[pl.ds(off + r - (_LANES - 1), _LANES)] = res

            return jnp.where(j == _LANES - 1, zero, res)

        lax.fori_loop(0, _CHUNK, rbody, zero)

    def body(i, carry):
        for buf, sem in ((buf0, sem0), (buf1, sem1)):
            k = 2 * i + (0 if buf is buf0 else 1)
            pltpu.make_async_copy(
                emb_hbm.at[pl.ds(0, _CHUNK * _D)], buf, sem).wait()
            compute(buf, chunk_off(k))

            @pl.when(k + 2 < _NCHUNKS)
            def _():
                pltpu.async_copy(
                    emb_hbm.at[
                        pl.ds((base_row + chunk_off(k + 2)) * _D, _CHUNK * _D)],
                    buf, sem)
        return carry

    lax.fori_loop(0, _NCHUNKS // 2, body, 0)
    pltpu.sync_copy(shard_v, s_hbm.at[pl.ds(base_row, _SHARD)])


def _sc_matvec(emb, w_flat):
    mesh = plsc.VectorSubcoreMesh(core_axis_name="c", subcore_axis_name="s")
    k = functools.partial(
        pl.kernel,
        mesh=mesh,
        out_type=jax.ShapeDtypeStruct((_V,), jnp.float32),
        scratch_types=[
            pltpu.VMEM((_D,), jnp.float32),
            pltpu.VMEM((_CHUNK * _D,), jnp.float32),
            pltpu.VMEM((_CHUNK * _D,), jnp.float32),
            pltpu.VMEM((_SHARD,), jnp.float32),
            pltpu.SemaphoreType.DMA,
            pltpu.SemaphoreType.DMA,
        ],
    )(_sc_mv_body)
    return k(emb, w_flat)


def _sc_pool_body(text_hbm, s_hbm, out_hbm, idx2_v, idx_v, val_v, res_v, sem):
    wid = lax.axis_index("s") * _NC + lax.axis_index("c")
    pltpu.sync_copy(text_hbm.at[:, pl.ds(wid * _BPW, _BPW)], idx2_v)

    def fbody(l, carry):
        for g in range(_BPW // _LANES):
            idx_v[pl.ds(l * _BPW + g * _LANES, _LANES)] = (
                idx2_v[l, pl.ds(g * _LANES, _LANES)])
        return carry

    lax.fori_loop(0, _L, fbody, 0)
    pltpu.async_copy(s_hbm.at[idx_v], val_v, sem).wait()
    ngroups = _BPW // _LANES
    zero = jnp.zeros((_LANES,), jnp.float32)

    def body(l, accs):
        row = l * _BPW
        return tuple(accs[g] + val_v[pl.ds(row + g * _LANES, _LANES)]
                     for g in range(ngroups))

    accs = lax.fori_loop(0, _L, body, tuple(zero for _ in range(ngroups)))
    for g in range(ngroups):
        res_v[pl.ds(g * _LANES, _LANES)] = accs[g]
    pltpu.sync_copy(res_v, out_hbm.at[pl.ds(wid * _BPW, _BPW)])


def _sc_pool(text, s):
    mesh = plsc.VectorSubcoreMesh(core_axis_name="c", subcore_axis_name="s")
    k = functools.partial(
        pl.kernel,
        mesh=mesh,
        out_type=jax.ShapeDtypeStruct((_B,), jnp.float32),
        scratch_types=[
            pltpu.VMEM((_L, _BPW), jnp.int32),
            pltpu.VMEM((_L * _BPW,), jnp.int32),
            pltpu.VMEM((_L * _BPW,), jnp.float32),
            pltpu.VMEM((_BPW,), jnp.float32),
            pltpu.SemaphoreType.DMA,
        ],
    )(_sc_pool_body)
    return k(text, s)


def kernel(text, text_len, emb, W, b):
    del text_len
    w_flat = (W * (1.0 / _L)).reshape(_D)
    s = _sc_matvec(emb.reshape(_V * _D), w_flat)
    pooled = _sc_pool(text, s)
    return pooled.reshape(_B, 1) + b


# speedup vs baseline: 1.4366x; 1.4366x over previous
import functools

import jax
import jax.numpy as jnp
from jax import lax
from jax.experimental import pallas as pl
from jax.experimental.pallas import tpu as pltpu
from jax.experimental.pallas import tpu_sc as plsc

_L = 200
_B = 4096
_V = 1_000_000
_D = 64

_PACK = 8
_KDIM = _PACK * _D
_ROWS = _V // _PACK
_BLK = 1024

_NC, _NS = 2, 16
_NW = _NC * _NS
_BPW = _B // _NW
_LANES = 16


def _matvec_body(x_ref, w_ref, o_ref):
    o_ref[...] = jnp.dot(x_ref[...].astype(jnp.bfloat16), w_ref[...],
                         preferred_element_type=jnp.float32)


def _scalarize(embr, wbd):
    grid = (_ROWS + _BLK - 1) // _BLK
    return pl.pallas_call(
        _matvec_body,
        grid=(grid,),
        in_specs=[
            pl.BlockSpec((_BLK, _KDIM), lambda i: (i, 0)),
            pl.BlockSpec((_KDIM, _PACK), lambda i: (0, 0)),
        ],
        out_specs=pl.BlockSpec((_BLK, _PACK), lambda i: (i, 0)),
        out_shape=jax.ShapeDtypeStruct((_ROWS, _PACK), jnp.float32),
    )(embr, wbd)


def _sc_pool_body(text_hbm, s_hbm, out_hbm, idx2_v, idx_v, val_v, res_v, sem):
    wid = lax.axis_index("s") * _NC + lax.axis_index("c")
    pltpu.sync_copy(text_hbm.at[:, pl.ds(wid * _BPW, _BPW)], idx2_v)

    def fbody(l, carry):
        for g in range(_BPW // _LANES):
            idx_v[pl.ds(l * _BPW + g * _LANES, _LANES)] = (
                idx2_v[l, pl.ds(g * _LANES, _LANES)])
        return carry

    lax.fori_loop(0, _L, fbody, 0)
    pltpu.async_copy(s_hbm.at[idx_v], val_v, sem).wait()
    ngroups = _BPW // _LANES
    zero = jnp.zeros((_LANES,), jnp.float32)

    def body(l, accs):
        row = l * _BPW
        return tuple(accs[g] + val_v[pl.ds(row + g * _LANES, _LANES)]
                     for g in range(ngroups))

    accs = lax.fori_loop(0, _L, body, tuple(zero for _ in range(ngroups)))
    for g in range(ngroups):
        res_v[pl.ds(g * _LANES, _LANES)] = accs[g]
    pltpu.sync_copy(res_v, out_hbm.at[pl.ds(wid * _BPW, _BPW)])


def _sc_pool(text, s):
    mesh = plsc.VectorSubcoreMesh(core_axis_name="c", subcore_axis_name="s")
    k = functools.partial(
        pl.kernel,
        mesh=mesh,
        out_type=jax.ShapeDtypeStruct((_B,), jnp.float32),
        scratch_types=[
            pltpu.VMEM((_L, _BPW), jnp.int32),
            pltpu.VMEM((_L * _BPW,), jnp.int32),
            pltpu.VMEM((_L * _BPW,), jnp.float32),
            pltpu.VMEM((_BPW,), jnp.float32),
            pltpu.SemaphoreType.DMA,
        ],
    )(_sc_pool_body)
    return k(text, s)


def kernel(text, text_len, emb, W, b):
    del text_len
    embr = emb.reshape(_ROWS, _KDIM)
    wbd = jnp.kron(jnp.eye(_PACK, dtype=jnp.float32),
                   W * (1.0 / _L)).astype(jnp.bfloat16)
    s = _scalarize(embr, wbd).reshape(_V)
    pooled = _sc_pool(text, s)
    return pooled.reshape(_B, 1) + b


# speedup vs baseline: 1.6099x; 1.1206x over previous
import functools

import jax
import jax.numpy as jnp
from jax import lax
from jax.experimental import pallas as pl
from jax.experimental.pallas import tpu as pltpu
from jax.experimental.pallas import tpu_sc as plsc

_L = 200
_B = 4096
_V = 1_000_000
_D = 64

_PACK = 8
_KDIM = _PACK * _D
_ROWS = _V // _PACK
_BLK = 5000

_NC, _NS = 2, 16
_NW = _NC * _NS
_BPW = _B // _NW
_LANES = 16


def _matvec_body(x_ref, w_ref, o_ref):
    o_ref[...] = jnp.dot(x_ref[...].astype(jnp.bfloat16), w_ref[...],
                         preferred_element_type=jnp.float32)


def _scalarize(embr, wbd):
    grid = _ROWS // _BLK
    return pl.pallas_call(
        _matvec_body,
        grid=(grid,),
        in_specs=[
            pl.BlockSpec((_BLK, _KDIM), lambda i: (i, 0)),
            pl.BlockSpec((_KDIM, 16 * _PACK), lambda i: (0, 0)),
        ],
        out_specs=pl.BlockSpec((_BLK, 16 * _PACK), lambda i: (i, 0)),
        out_shape=jax.ShapeDtypeStruct((_ROWS, 16 * _PACK), jnp.float32),
    )(embr, wbd)


def _sc_pool_body(text_hbm, s_hbm, out_hbm, idx2_v, idx_v, val_v, res_v, sem):
    wid = lax.axis_index("s") * _NC + lax.axis_index("c")
    pltpu.sync_copy(text_hbm.at[:, pl.ds(wid * _BPW, _BPW)], idx2_v)

    def fbody(l, carry):
        for g in range(_BPW // _LANES):
            v = idx2_v[l, pl.ds(g * _LANES, _LANES)]
            idx_v[pl.ds(l * _BPW + g * _LANES, _LANES)] = (
                ((v >> 3) << 7) + (v & 7))
        return carry

    lax.fori_loop(0, _L, fbody, 0)
    pltpu.async_copy(s_hbm.at[idx_v], val_v, sem).wait()
    ngroups = _BPW // _LANES
    zero = jnp.zeros((_LANES,), jnp.float32)

    def body(l, accs):
        row = l * _BPW
        return tuple(accs[g] + val_v[pl.ds(row + g * _LANES, _LANES)]
                     for g in range(ngroups))

    accs = lax.fori_loop(0, _L, body, tuple(zero for _ in range(ngroups)))
    for g in range(ngroups):
        res_v[pl.ds(g * _LANES, _LANES)] = accs[g]
    pltpu.sync_copy(res_v, out_hbm.at[pl.ds(wid * _BPW, _BPW)])


def _sc_pool(text, s):
    mesh = plsc.VectorSubcoreMesh(core_axis_name="c", subcore_axis_name="s")
    k = functools.partial(
        pl.kernel,
        mesh=mesh,
        out_type=jax.ShapeDtypeStruct((_B,), jnp.float32),
        scratch_types=[
            pltpu.VMEM((_L, _BPW), jnp.int32),
            pltpu.VMEM((_L * _BPW,), jnp.int32),
            pltpu.VMEM((_L * _BPW,), jnp.float32),
            pltpu.VMEM((_BPW,), jnp.float32),
            pltpu.SemaphoreType.DMA,
        ],
    )(_sc_pool_body)
    return k(text, s)


def kernel(text, text_len, emb, W, b):
    del text_len
    embr = emb.reshape(_ROWS, _KDIM)
    wbd = jnp.tile(jnp.kron(jnp.eye(_PACK, dtype=jnp.float32),
                            W * (1.0 / _L)), (1, 16)).astype(jnp.bfloat16)
    s = _scalarize(embr, wbd).reshape(16 * _V)
    pooled = _sc_pool(text, s)
    return pooled.reshape(_B, 1) + b
